# baseline (speedup 1.0000x reference)
import functools

import jax
import jax.numpy as jnp
from jax import lax
from jax.experimental import pallas as pl
from jax.experimental.pallas import tpu as pltpu
from jax.experimental.pallas import tpu_sc as plsc

_B = 1024
_D = 1024
_NP = 16
_NK = 32
_NL = 4
_Q = _NP * _NL
_PL = (_NK - 1) * _NL + 1

_NW = 32
_BPW = _B // _NW
_C = 16
_NSEM = 8
_NCH = _BPW * _Q // _C
_HPB = _Q // _C


def _topk_body(q_ref, k_ref, idx_ref):
    q = q_ref[...]
    k = k_ref[...]
    dots = lax.dot_general(k, q, (((1,), (1,)), ((), ())),
                           preferred_element_type=jnp.float32)
    kn = jnp.sum(k * k, axis=1, keepdims=True)
    score = dots * lax.rsqrt(kn)
    score = jnp.where(score != score, jnp.inf, score)
    m_id = lax.broadcasted_iota(jnp.int32, (_NK, _B), 0)
    rank_rows = []
    for j in range(_NK):
        sj = score[j:j + 1, :]
        beats = (score > sj) | ((score == sj) & (m_id < j))
        rank_rows.append(jnp.sum(beats.astype(jnp.int32), axis=0,
                                 keepdims=True))
    rank = jnp.concatenate(rank_rows, axis=0)
    row4 = 4 * m_id
    for qq in range(_Q):
        local = jnp.where(m_id == 0, 0, row4 + (qq % _NL - 3))
        sel = jnp.sum(jnp.where(rank == qq // _NL, local, 0),
                      axis=0, keepdims=True)
        idx_ref[qq:qq + 1, :] = sel


_topk_call = pl.pallas_call(
    _topk_body,
    out_shape=jax.ShapeDtypeStruct((_Q, _B), jnp.int32),
)


def _gather_body(pool_hbm, idx_hbm, out_hbm, idx_v, pool_v, wsems):
    nc = 2
    wid = lax.axis_index("s") * nc + lax.axis_index("c")
    for q in range(_Q):
        pltpu.async_copy(idx_hbm.at[pl.ds(q * _B + wid * _BPW, _BPW)],
                         idx_v.at[pl.ds(q * _BPW, _BPW)], wsems[0])
    pltpu.sync_copy(pool_hbm.at[pl.ds(0, _PL * _D)], pool_v)
    pltpu.make_async_copy(idx_hbm.at[pl.ds(0, _Q * _BPW)], idx_v,
                          wsems[0]).wait()

    def wait_write(s):
        pltpu.make_async_copy(pool_hbm.at[pl.ds(0, _D)],
                              pool_v.at[pl.ds(0, _D)], wsems[s]).wait()

    def body(q, carry):
        for half in range(_BPW // 16):
            rows = idx_v[pl.ds(q * _BPW + half * 16, 16)]
            for l in range(16):
                s = l % _NSEM
                bb = half * 16 + l
                if l < _NSEM and half == 0:
                    @pl.when(q > 0)
                    def _():
                        wait_write(s)
                else:
                    wait_write(s)

                row = rows[l]
                src = pool_v.at[pl.ds(row * _D, _D)]
                dst = out_hbm.at[wid * _BPW + bb, q]
                pltpu.async_copy(src, dst, wsems[s])
        return carry

    lax.fori_loop(0, _Q, body, 0)
    for s in range(_NSEM):
        wait_write(s)


@functools.cache
def _make_gather_call():
    mesh = plsc.VectorSubcoreMesh(core_axis_name="c", subcore_axis_name="s")
    return pl.kernel(
        _gather_body,
        out_type=jax.ShapeDtypeStruct((_B, _Q, _D), jnp.float32),
        mesh=mesh,
        scratch_types=[
            pltpu.VMEM((_Q * _BPW,), jnp.int32),
            pltpu.VMEM((_PL * _D,), jnp.float32),
            [pltpu.SemaphoreType.DMA] * _NSEM,
        ],
    )


def kernel(query, prompt_pool, prompt_key):
    idx4_t = _topk_call(query, prompt_key)
    pool_rows = prompt_pool.reshape(_NK * _NL, _D)
    pool_c = jnp.concatenate(
        [jnp.zeros((1, _D), jnp.float32), pool_rows[_NL:]], axis=0)
    return _make_gather_call()(pool_c.reshape(-1),
                               idx4_t.reshape(-1))


# speedup vs baseline: 1.0231x; 1.0231x over previous
import functools

import jax
import jax.numpy as jnp
from jax import lax
from jax.experimental import pallas as pl
from jax.experimental.pallas import tpu as pltpu
from jax.experimental.pallas import tpu_sc as plsc

_B = 1024
_D = 1024
_NP = 16
_NK = 32
_NL = 4
_Q = _NP * _NL
_PL = (_NK - 1) * _NL + 1

_NW = 32
_BPW = _B // _NW
_C = 16
_NSEM = 16
_NCH = _BPW * _Q // _C
_HPB = _Q // _C


def _topk_body(q_ref, k_ref, idx_ref):
    q = q_ref[...]
    k = k_ref[...]
    dots = lax.dot_general(k, q, (((1,), (1,)), ((), ())),
                           preferred_element_type=jnp.float32)
    kn = jnp.sum(k * k, axis=1, keepdims=True)
    score = dots * lax.rsqrt(kn)
    score = jnp.where(score != score, jnp.inf, score)
    m_id = lax.broadcasted_iota(jnp.int32, (_NK, _B), 0)
    rank_rows = []
    for j in range(_NK):
        sj = score[j:j + 1, :]
        beats = (score > sj) | ((score == sj) & (m_id < j))
        rank_rows.append(jnp.sum(beats.astype(jnp.int32), axis=0,
                                 keepdims=True))
    rank = jnp.concatenate(rank_rows, axis=0)
    row4 = 4 * m_id
    for qq in range(_Q):
        local = jnp.where(m_id == 0, 0, row4 + (qq % _NL - 3))
        sel = jnp.sum(jnp.where(rank == qq // _NL, local, 0),
                      axis=0, keepdims=True)
        idx_ref[qq:qq + 1, :] = sel


_topk_call = pl.pallas_call(
    _topk_body,
    out_shape=jax.ShapeDtypeStruct((_Q, _B), jnp.int32),
)


def _gather_body(pool_hbm, idx_hbm, out_hbm, idx_v, pool_v, wsems):
    nc = 2
    wid = lax.axis_index("s") * nc + lax.axis_index("c")
    pltpu.sync_copy(idx_hbm.at[pl.ds(wid * _BPW * _Q, _BPW * _Q)], idx_v)
    pltpu.sync_copy(pool_hbm.at[pl.ds(0, _PL * _D)], pool_v)

    def wait_write(s):
        pltpu.make_async_copy(pool_hbm.at[pl.ds(0, _D)],
                              pool_v.at[pl.ds(0, _D)], wsems[s]).wait()

    def body(g, carry):
        rows = idx_v[pl.ds(g * 16, 16)]
        for l in range(16):
            s = l % _NSEM
            if l < _NSEM:
                @pl.when(g > 0)
                def _():
                    wait_write(s)
            else:
                wait_write(s)

            r = g * 16 + l
            row = rows[l]
            b = wid * _BPW + r // _Q
            q = r % _Q
            src = pool_v.at[pl.ds(row * _D, _D)]
            dst = out_hbm.at[b, q]
            pltpu.async_copy(src, dst, wsems[s])
        return carry

    lax.fori_loop(0, _BPW * _Q // 16, body, 0)
    for s in range(_NSEM):
        wait_write(s)


@functools.cache
def _make_gather_call():
    mesh = plsc.VectorSubcoreMesh(core_axis_name="c", subcore_axis_name="s")
    return pl.kernel(
        _gather_body,
        out_type=jax.ShapeDtypeStruct((_B, _Q, _D), jnp.float32),
        mesh=mesh,
        scratch_types=[
            pltpu.VMEM((_BPW * _Q,), jnp.int32),
            pltpu.VMEM((_PL * _D,), jnp.float32),
            [pltpu.SemaphoreType.DMA] * _NSEM,
        ],
    )


def kernel(query, prompt_pool, prompt_key):
    idx4_t = _topk_call(query, prompt_key)
    idx4 = idx4_t.T.reshape(-1)
    pool_rows = prompt_pool.reshape(_NK * _NL, _D)
    pool_c = jnp.concatenate(
        [jnp.zeros((1, _D), jnp.float32), pool_rows[_NL:]], axis=0)
    return _make_gather_call()(pool_c.reshape(-1), idx4)
